# Initial kernel scaffold; baseline (speedup 1.0000x reference)
import functools
import jax
import jax.numpy as jnp
from jax import lax
from jax.experimental import pallas as pl
from jax.experimental.pallas import tpu as pltpu
from jax.experimental.pallas import tpu_sc as plsc

MIN_NORM = 1e-5
EPS = 1e-7
MAX_NORM = 1e6

N = 10000
E = 320000
D = 128
NPAD = 10240
BR = 256
NTILES = 32
EPT = E // NTILES
ROWS_PT = NPAD // 16
NEG = -3.0e38


def _masks(r):
    col = lax.broadcasted_iota(jnp.int32, (r, D), 1)
    is0 = col == 0
    m1 = jnp.where(is0, 0.0, 1.0)
    return is0, m1


def _cosh(x):
    return jnp.cosh(jnp.clip(x, -15.0, 15.0))


def _sinh(x):
    return jnp.sinh(jnp.clip(x, -15.0, 15.0))


def _head(z, is0):
    return jnp.sum(jnp.where(is0, z, 0.0), axis=1, keepdims=True)


def _proj(z, K, is0, m1):
    y = z * m1
    ysq = jnp.sum(y * y, axis=1, keepdims=True)
    head = jnp.sqrt(jnp.maximum(K + ysq, EPS))
    return jnp.where(is0, head, z)


def _expmap0(u, sqrtK, K, is0, m1):
    xt = u * m1
    xn = jnp.maximum(jnp.sqrt(jnp.sum(xt * xt, axis=1, keepdims=True)), MIN_NORM)
    theta = xn / sqrtK
    head = sqrtK * _cosh(theta)
    tail = sqrtK * _sinh(theta) * xt / xn
    z = jnp.where(is0, head, tail)
    return _proj(z, K, is0, m1)


def _logmap0(z, sqrtK, is0, m1):
    y = z * m1
    yn = jnp.maximum(jnp.sqrt(jnp.sum(y * y, axis=1, keepdims=True)), MIN_NORM)
    x0 = _head(z, is0)
    theta = jnp.maximum(x0 / sqrtK, 1.0 + EPS)
    arc = jnp.log(theta + jnp.sqrt(jnp.maximum((theta - 1.0) * (theta + 1.0), 0.0)))
    return sqrtK * arc * y / yn


def _expmap(u, x, sqrtK, K, is0, m1):
    u0 = _head(u, is0)
    dot = jnp.sum(u * u, axis=1, keepdims=True) - 2.0 * u0 * u0
    normu = jnp.sqrt(jnp.maximum(dot, EPS))
    normu = jnp.minimum(normu, MAX_NORM)
    theta = jnp.maximum(normu / sqrtK, MIN_NORM)
    result = _cosh(theta) * x + _sinh(theta) * u / theta
    return _proj(result, K, is0, m1)


def _mobius_add_bias(res, hb, sqrtK, K, is0, m1):
    u = _logmap0(hb, sqrtK, is0, m1)
    x0 = _head(res, is0)
    y = res * m1
    yn = jnp.maximum(jnp.sqrt(jnp.sum(y * y, axis=1, keepdims=True)), MIN_NORM)
    y_n = y / yn
    v = jnp.where(is0, -yn, (sqrtK - x0) * y_n)
    alpha = jnp.sum(y_n * u, axis=1, keepdims=True) / sqrtK
    r = u - alpha * v
    ux = jnp.sum(res * r * m1, axis=1, keepdims=True)
    head = ux / jnp.maximum(x0, EPS)
    r = jnp.where(is0, head, r)
    return _expmap(r, res, sqrtK, K, is0, m1)


def _dense_prelude(xh, W, brow, linWT, sqrtK, is0, m1):
    K = sqrtK * sqrtK
    u = _logmap0(xh, sqrtK, is0, m1)
    mvt = jnp.dot(u, W, preferred_element_type=jnp.float32)
    res = _expmap0(mvt, sqrtK, K, is0, m1)
    res = _proj(res, K, is0, m1)
    is0r, m1r = _masks(1)
    hb = _proj(_expmap0(brow * m1r, sqrtK, K, is0r, m1r), K, is0r, m1r)
    res = _proj(_mobius_add_bias(res, hb, sqrtK, K, is0, m1), K, is0, m1)
    x_tan = _logmap0(res, sqrtK, is0, m1)
    xs = jnp.dot(x_tan, linWT, preferred_element_type=jnp.float32)
    a = jnp.sum(xs, axis=-1)
    return res, xs, a


def _block_final(res, agg, sqrtK_in, sqrtK_out, is0, m1):
    K_in = sqrtK_in * sqrtK_in
    K_out = sqrtK_out * sqrtK_out
    out = _proj(_expmap(agg, res, sqrtK_in, K_in, is0, m1), K_in, is0, m1)
    out = jax.nn.relu(_logmap0(out, sqrtK_in, is0, m1))
    out = out * m1
    return _proj(_expmap0(out, sqrtK_out, K_out, is0, m1), K_out, is0, m1)


def _tc_first_body(x_ref, W_ref, b_ref, linWT_ref, s_ref, res_ref, xs_ref, a_ref):
    is0, m1 = _masks(BR)
    sqrtK = s_ref[0, 0]
    K = sqrtK * sqrtK
    xh = _proj(_expmap0(x_ref[...] * m1, sqrtK, K, is0, m1), K, is0, m1)
    res, xs, a = _dense_prelude(xh, W_ref[...], b_ref[...], linWT_ref[...], sqrtK, is0, m1)
    res_ref[...] = res
    xs_ref[...] = xs
    a_ref[...] = a


def _tc_comb_body(mp_ref, a_ref, out_ref):
    m = jnp.max(mp_ref[...], axis=0)
    out_ref[...] = jnp.maximum(m, a_ref[...])


def _tc_mid_body(res_ref, xs_ref, a_ref, M_ref, S_ref, dp_ref, W_ref, b_ref,
                 linWT_ref, s_ref, res2_ref, xs2_ref, a2_ref):
    is0, m1 = _masks(BR)
    sqrtK_in = s_ref[0, 0]
    sqrtK_out = s_ref[0, 1]
    xs = xs_ref[...]
    ex_self = jnp.exp(a_ref[...] - M_ref[...])[:, None]
    den = (jnp.sum(dp_ref[...], axis=0) + 1e-16)[:, None] + ex_self
    num = S_ref[0] + S_ref[1] + ex_self * xs
    agg = num / den
    out = _block_final(res_ref[...], agg, sqrtK_in, sqrtK_out, is0, m1)
    res, xs2, a2 = _dense_prelude(out, W_ref[...], b_ref[...], linWT_ref[...], sqrtK_out, is0, m1)
    res2_ref[...] = res
    xs2_ref[...] = xs2
    a2_ref[...] = a2


def _tc_last_body(res_ref, xs_ref, a_ref, M_ref, S_ref, dp_ref, s_ref, out_ref):
    is0, m1 = _masks(BR)
    sqrtK_in = s_ref[0, 0]
    sqrtK_out = s_ref[0, 1]
    xs = xs_ref[...]
    ex_self = jnp.exp(a_ref[...] - M_ref[...])[:, None]
    den = (jnp.sum(dp_ref[...], axis=0) + 1e-16)[:, None] + ex_self
    num = S_ref[0] + S_ref[1] + ex_self * xs
    agg = num / den
    out_ref[...] = _block_final(res_ref[...], agg, sqrtK_in, sqrtK_out, is0, m1)


_GRID = NPAD // BR
_f32 = jnp.float32

_rowspec = pl.BlockSpec((BR, D), lambda i: (i, 0))
_vecspec = pl.BlockSpec((BR,), lambda i: (i,))
_wspec = pl.BlockSpec((D, D), lambda i: (0, 0))
_bspec = pl.BlockSpec((1, D), lambda i: (0, 0))
_sspec = pl.BlockSpec((1, 2), lambda i: (0, 0))
_s1spec = pl.BlockSpec((1, 1), lambda i: (0, 0))
_Sspec = pl.BlockSpec((2, BR, D), lambda i: (0, i, 0))
_dpspec = pl.BlockSpec((NTILES, BR), lambda i: (0, i))


def _tc_first(x_pad, W, brow, linWT, scal):
    return pl.pallas_call(
        _tc_first_body,
        grid=(_GRID,),
        in_specs=[_rowspec, _wspec, _bspec, _wspec, _s1spec],
        out_specs=[_rowspec, _rowspec, _vecspec],
        out_shape=[jax.ShapeDtypeStruct((NPAD, D), _f32),
                   jax.ShapeDtypeStruct((NPAD, D), _f32),
                   jax.ShapeDtypeStruct((NPAD,), _f32)],
    )(x_pad, W, brow, linWT, scal)


def _tc_comb(mp, a):
    CB = 2048
    return pl.pallas_call(
        _tc_comb_body,
        grid=(NPAD // CB,),
        in_specs=[pl.BlockSpec((NTILES, CB), lambda i: (0, i)),
                  pl.BlockSpec((CB,), lambda i: (i,))],
        out_specs=pl.BlockSpec((CB,), lambda i: (i,)),
        out_shape=jax.ShapeDtypeStruct((NPAD,), _f32),
    )(mp, a)


def _tc_mid(res, xs, a, M, S, dp, W, brow, linWT, scal):
    return pl.pallas_call(
        _tc_mid_body,
        grid=(_GRID,),
        in_specs=[_rowspec, _rowspec, _vecspec, _vecspec, _Sspec, _dpspec,
                  _wspec, _bspec, _wspec, _sspec],
        out_specs=[_rowspec, _rowspec, _vecspec],
        out_shape=[jax.ShapeDtypeStruct((NPAD, D), _f32),
                   jax.ShapeDtypeStruct((NPAD, D), _f32),
                   jax.ShapeDtypeStruct((NPAD,), _f32)],
    )(res, xs, a, M, S, dp, W, brow, linWT, scal)


def _tc_last(res, xs, a, M, S, dp, scal):
    return pl.pallas_call(
        _tc_last_body,
        grid=(_GRID,),
        in_specs=[_rowspec, _rowspec, _vecspec, _vecspec, _Sspec, _dpspec, _sspec],
        out_specs=_rowspec,
        out_shape=jax.ShapeDtypeStruct((NPAD, D), _f32),
    )(res, xs, a, M, S, dp, scal)


_mesh = plsc.VectorSubcoreMesh(core_axis_name="c", subcore_axis_name="s")


@functools.partial(
    pl.kernel,
    out_type=jax.ShapeDtypeStruct((NTILES, NPAD), _f32),
    mesh=_mesh,
    scratch_types=[
        pltpu.VMEM((NPAD,), _f32),
        pltpu.VMEM((NPAD,), _f32),
        pltpu.VMEM((EPT,), jnp.int32),
        pltpu.VMEM((EPT,), jnp.int32),
    ],
)
def _sc_max(src_hbm, dst_hbm, a_hbm, out_hbm, a_v, m_v, s_v, d_v):
    cid = lax.axis_index("c")
    sid = lax.axis_index("s")
    wid = sid * 2 + cid
    base = wid * EPT
    pltpu.sync_copy(a_hbm, a_v)
    pltpu.sync_copy(src_hbm.at[pl.ds(base, EPT)], s_v)
    pltpu.sync_copy(dst_hbm.at[pl.ds(base, EPT)], d_v)

    def init_body(i, carry):
        m_v[pl.ds(i * 16, 16)] = jnp.full((16,), NEG, _f32)
        return carry

    lax.fori_loop(0, NPAD // 16, init_body, 0)

    def edge_body(i, carry):
        s = s_v[pl.ds(i * 16, 16)]
        d = d_v[pl.ds(i * 16, 16)]
        g = plsc.load_gather(a_v, [s])
        ok = s != d
        g = jnp.where(ok, g, NEG)
        need = jnp.logical_and(ok, plsc.load_gather(m_v, [d]) < g)

        def wcond(need):
            return jnp.any(need)

        def wbody(need):
            plsc.store_scatter(m_v, [d], g, mask=need)
            cur = plsc.load_gather(m_v, [d])
            return jnp.logical_and(need, cur < g)

        lax.while_loop(wcond, wbody, need)
        return carry

    lax.fori_loop(0, EPT // 16, edge_body, 0)
    pltpu.sync_copy(m_v, out_hbm.at[wid])


@functools.partial(
    pl.kernel,
    out_type=(jax.ShapeDtypeStruct((2, NPAD, D), _f32),
              jax.ShapeDtypeStruct((NTILES, NPAD), _f32)),
    mesh=_mesh,
    scratch_types=[
        pltpu.VMEM((NPAD,), _f32),
        pltpu.VMEM((NPAD,), _f32),
        pltpu.VMEM((NPAD,), _f32),
        pltpu.VMEM((EPT,), jnp.int32),
        pltpu.VMEM((EPT,), jnp.int32),
        pltpu.VMEM((16, D), _f32),
        pltpu.VMEM((16, D), _f32),
        pltpu.VMEM((16,), _f32),
        pltpu.VMEM_SHARED((NPAD, D), _f32),
        pltpu.SemaphoreType.DMA,
    ],
)
def _sc_heavy(src_hbm, dst_hbm, a_hbm, M_hbm, xs_hbm, S_hbm, den_hbm,
              a_v, m_v, den_v, s_v, d_v, rows_v, srow_v, ex_v, S_sh, sem):
    cid = lax.axis_index("c")
    sid = lax.axis_index("s")
    wid = sid * 2 + cid
    base = wid * EPT
    pltpu.sync_copy(a_hbm, a_v)
    pltpu.sync_copy(M_hbm, m_v)
    pltpu.sync_copy(src_hbm.at[pl.ds(base, EPT)], s_v)
    pltpu.sync_copy(dst_hbm.at[pl.ds(base, EPT)], d_v)

    def zden(i, carry):
        den_v[pl.ds(i * 16, 16)] = jnp.zeros((16,), _f32)
        return carry

    lax.fori_loop(0, NPAD // 16, zden, 0)
    for j in range(16):
        for cb in range(D // 16):
            rows_v[j, pl.ds(cb * 16, 16)] = jnp.zeros((16,), _f32)

    def zsh(i, carry):
        pltpu.sync_copy(rows_v, S_sh.at[pl.ds(sid * ROWS_PT + i * 16, 16)])
        return carry

    lax.fori_loop(0, ROWS_PT // 16, zsh, 0)
    plsc.subcore_barrier()

    def edge_body(i, carry):
        s = s_v[pl.ds(i * 16, 16)]
        d = d_v[pl.ds(i * 16, 16)]
        g = plsc.load_gather(a_v, [s])
        m = plsc.load_gather(m_v, [d])
        ok = s != d
        ex = jnp.where(ok, jnp.exp(g - m), 0.0)
        plsc.addupdate_scatter(den_v, [d], ex)
        pltpu.async_copy(xs_hbm.at[s], rows_v, sem).wait()
        ex_v[...] = ex
        for j in range(16):
            e = ex_v[j]
            for cb in range(D // 16):
                srow_v[j, pl.ds(cb * 16, 16)] = rows_v[j, pl.ds(cb * 16, 16)] * e
        pltpu.sync_copy(srow_v, S_sh.at[d], add=True)
        return carry

    lax.fori_loop(0, EPT // 16, edge_body, 0)
    pltpu.sync_copy(den_v, den_hbm.at[wid])
    plsc.subcore_barrier()
    pltpu.sync_copy(S_sh.at[pl.ds(sid * ROWS_PT, ROWS_PT)],
                    S_hbm.at[cid, pl.ds(sid * ROWS_PT, ROWS_PT)])


def kernel(x, edge_index, W0, b0, lin0, W1, b1, lin1, c0, c1, c2):
    n = x.shape[0]
    src = edge_index[0]
    dst = edge_index[1]

    cin0 = jax.nn.softplus(c0[0])
    cin1 = jax.nn.softplus(c1[0])
    cin2 = jax.nn.softplus(c2[0])
    sq0 = 1.0 / jnp.sqrt(cin0)
    sq1 = 1.0 / jnp.sqrt(cin1)
    sq2 = 1.0 / jnp.sqrt(cin2)

    x_pad = jnp.zeros((NPAD, D), _f32).at[:n, 1:].set(x)
    s0 = jnp.reshape(sq0, (1, 1))
    s01 = jnp.stack([sq0, sq1]).reshape(1, 2)
    s12 = jnp.stack([sq1, sq2]).reshape(1, 2)

    res0, xs0, a0 = _tc_first(x_pad, W0, b0.reshape(1, D), lin0.T, s0)
    mp0 = _sc_max(src, dst, a0)
    M0 = _tc_comb(mp0, a0)
    S0, dp0 = _sc_heavy(src, dst, a0, M0, xs0)

    res1, xs1, a1 = _tc_mid(res0, xs0, a0, M0, S0, dp0,
                            W1, b1.reshape(1, D), lin1.T, s01)
    mp1 = _sc_max(src, dst, a1)
    M1 = _tc_comb(mp1, a1)
    S1, dp1 = _sc_heavy(src, dst, a1, M1, xs1)

    out = _tc_last(res1, xs1, a1, M1, S1, dp1, s12)
    return out[:n]


# speedup vs baseline: 1.7678x; 1.7678x over previous
import functools
import jax
import jax.numpy as jnp
from jax import lax
from jax.experimental import pallas as pl
from jax.experimental.pallas import tpu as pltpu
from jax.experimental.pallas import tpu_sc as plsc

MIN_NORM = 1e-5
EPS = 1e-7
MAX_NORM = 1e6

N = 10000
E = 320000
D = 128
NPAD = 10240
BR = 256
NTILES = 32
EPT = E // NTILES
ROWS_PT = NPAD // 16
ECHUNK = 2000
NEG = -3.0e38

_f32 = jnp.float32


def _r_cosh(x):
    return jnp.cosh(jnp.clip(x, -15.0, 15.0))


def _r_sinh(x):
    return jnp.sinh(jnp.clip(x, -15.0, 15.0))


def _r_proj(x, c):
    K = 1.0 / c
    y = x[:, 1:]
    y_sqnorm = jnp.sum(y * y, axis=1, keepdims=True)
    head = jnp.sqrt(jnp.maximum(K + y_sqnorm, EPS))
    return jnp.concatenate([head, y], axis=1)


def _r_expmap0(u, c):
    K = 1.0 / c
    sqrtK = jnp.sqrt(K)
    xt = u[:, 1:]
    x_norm = jnp.maximum(jnp.sqrt(jnp.sum(xt * xt, axis=1, keepdims=True)), MIN_NORM)
    theta = x_norm / sqrtK
    head = sqrtK * _r_cosh(theta)
    tail = sqrtK * _r_sinh(theta) * xt / x_norm
    return _r_proj(jnp.concatenate([head * jnp.ones_like(x_norm), tail], axis=1), c)


def _r_logmap0(x, c):
    K = 1.0 / c
    sqrtK = jnp.sqrt(K)
    y = x[:, 1:]
    y_norm = jnp.maximum(jnp.sqrt(jnp.sum(y * y, axis=1, keepdims=True)), MIN_NORM)
    theta = jnp.maximum(x[:, 0:1] / sqrtK, 1.0 + EPS)
    tail = sqrtK * jnp.arccosh(jnp.maximum(theta, 1.0 + EPS)) * y / y_norm
    return jnp.concatenate([jnp.zeros_like(tail[:, :1]), tail], axis=1)


def _r_minkowski_norm(u):
    dot = jnp.sum(u * u, axis=-1, keepdims=True) - 2.0 * u[..., 0:1] * u[..., 0:1]
    return jnp.sqrt(jnp.maximum(dot, EPS))


def _r_expmap(u, x, c):
    K = 1.0 / c
    sqrtK = jnp.sqrt(K)
    normu = jnp.minimum(_r_minkowski_norm(u), MAX_NORM)
    theta = jnp.maximum(normu / sqrtK, MIN_NORM)
    result = _r_cosh(theta) * x + _r_sinh(theta) * u / theta
    return _r_proj(result, c)


def _r_proj_tan(u, x, c):
    ux = jnp.sum(x[:, 1:] * u[:, 1:], axis=1, keepdims=True)
    head = ux / jnp.maximum(x[:, 0:1], EPS)
    return jnp.concatenate([head, u[:, 1:]], axis=1)


def _r_ptransp0(x, u, c):
    K = 1.0 / c
    sqrtK = jnp.sqrt(K)
    x0 = x[:, 0:1]
    y = x[:, 1:]
    y_norm = jnp.maximum(jnp.sqrt(jnp.sum(y * y, axis=1, keepdims=True)), MIN_NORM)
    y_n = y / y_norm
    v = jnp.concatenate([-y_norm, (sqrtK - x0) * y_n], axis=1)
    alpha = jnp.sum(y_n * u[:, 1:], axis=1, keepdims=True) / sqrtK
    res = u - alpha * v
    return _r_proj_tan(res, x, c)


def _r_proj_tan0(u):
    return jnp.concatenate([jnp.zeros_like(u[:, 0:1]), u[:, 1:]], axis=1)


def _r_mobius_add(x, y, c):
    u = _r_logmap0(y, c)
    v = _r_ptransp0(x, u, c)
    return _r_expmap(v, x, c)


def _r_prelude(x, W, b, c_in):
    mv = _r_expmap0(_r_logmap0(x, c_in) @ W, c_in)
    res = _r_proj(mv, c_in)
    bias = _r_proj_tan0(b.reshape(1, -1))
    hyp_bias = _r_proj(_r_expmap0(bias, c_in), c_in)
    return _r_proj(_r_mobius_add(res, hyp_bias, c_in), c_in)


def _r_block0(x, src, dst, edge_mask, W, b, linW, c_in, c_out, n):
    res = _r_prelude(x, W, b, c_in)
    x_tan = _r_logmap0(res, c_in)
    xs = x_tan @ linW.T
    a = jnp.sum(xs, axis=-1)
    ae = jnp.where(edge_mask, a[src] + a[dst], -jnp.inf)
    amax = jax.ops.segment_max(ae, dst, num_segments=n)
    ex = jnp.exp(ae - amax[dst])
    denom = jax.ops.segment_sum(ex, dst, num_segments=n)
    alpha = ex / (denom[dst] + 1e-16)
    agg = jax.ops.segment_sum(alpha[:, None] * xs[src], dst, num_segments=n)
    out = _r_proj(_r_expmap(res, agg, c_in), c_in)
    out = jax.nn.relu(_r_logmap0(out, c_in))
    out = _r_proj_tan0(out)
    return _r_proj(_r_expmap0(out, c_out), c_out)


def _masks(r):
    col = lax.broadcasted_iota(jnp.int32, (r, D), 1)
    is0 = col == 0
    m1 = jnp.where(is0, 0.0, 1.0)
    return is0, m1


def _cosh(x):
    x = jnp.clip(x, -15.0, 15.0)
    return 0.5 * (jnp.exp(x) + jnp.exp(-x))


def _sinh(x):
    x = jnp.clip(x, -15.0, 15.0)
    return 0.5 * (jnp.exp(x) - jnp.exp(-x))


def _head(z, is0):
    return jnp.sum(jnp.where(is0, z, 0.0), axis=1, keepdims=True)


def _proj(z, K, is0, m1):
    y = z * m1
    ysq = jnp.sum(y * y, axis=1, keepdims=True)
    head = jnp.sqrt(jnp.maximum(K + ysq, EPS))
    return jnp.where(is0, head, z)


def _expmap0(u, sqrtK, K, is0, m1):
    xt = u * m1
    xn = jnp.maximum(jnp.sqrt(jnp.sum(xt * xt, axis=1, keepdims=True)), MIN_NORM)
    theta = xn / sqrtK
    head = sqrtK * _cosh(theta)
    tail = sqrtK * _sinh(theta) * xt / xn
    z = jnp.where(is0, head, tail)
    return _proj(z, K, is0, m1)


def _logmap0(z, sqrtK, is0, m1):
    y = z * m1
    yn = jnp.maximum(jnp.sqrt(jnp.sum(y * y, axis=1, keepdims=True)), MIN_NORM)
    x0 = _head(z, is0)
    theta = jnp.maximum(x0 / sqrtK, 1.0 + EPS)
    arc = jnp.log(theta + jnp.sqrt(jnp.maximum((theta - 1.0) * (theta + 1.0), 0.0)))
    return sqrtK * arc * y / yn


def _expmap(u, x, sqrtK, K, is0, m1):
    u0 = _head(u, is0)
    dot = jnp.sum(u * u, axis=1, keepdims=True) - 2.0 * u0 * u0
    normu = jnp.sqrt(jnp.maximum(dot, EPS))
    normu = jnp.minimum(normu, MAX_NORM)
    theta = jnp.maximum(normu / sqrtK, MIN_NORM)
    result = _cosh(theta) * x + _sinh(theta) * u / theta
    return _proj(result, K, is0, m1)


def _tc_xs_body(res_ref, linWT_ref, s_ref, xs_ref, a_ref):
    is0, m1 = _masks(BR)
    sqrtK = s_ref[0, 0]
    x_tan = _logmap0(res_ref[...], sqrtK, is0, m1)
    xs = jnp.dot(x_tan, linWT_ref[...], preferred_element_type=_f32)
    xs_ref[...] = xs
    a_ref[...] = jnp.sum(xs, axis=-1)


def _tc_comb_body(mp_ref, a_ref, out_ref):
    m = jnp.max(mp_ref[...], axis=0)
    out_ref[...] = jnp.maximum(m, a_ref[...])


def _tc_tail_body(res_ref, xs_ref, a_ref, M_ref, S_ref, dp_ref, th_ref, s_ref,
                  out_ref):
    is0, m1 = _masks(BR)
    sqrtK_in = s_ref[0, 0]
    sqrtK_out = s_ref[0, 1]
    K_in = sqrtK_in * sqrtK_in
    K_out = sqrtK_out * sqrtK_out
    xs = xs_ref[...]
    res = res_ref[...]
    ex_self = jnp.exp(a_ref[...] - M_ref[...])[:, None]
    den = (jnp.sum(dp_ref[...], axis=0) + 1e-16)[:, None] + ex_self
    num = S_ref[0] + S_ref[1] + ex_self * xs
    agg = num / den
    theta = th_ref[...][:, None]
    out = _cosh(theta) * agg + _sinh(theta) * res / theta
    out = _proj(out, K_in, is0, m1)
    out = _proj(out, K_in, is0, m1)
    out = jax.nn.relu(_logmap0(out, sqrtK_in, is0, m1)) * m1
    out_ref[...] = _proj(_expmap0(out, sqrtK_out, K_out, is0, m1), K_out, is0, m1)


_GRID = NPAD // BR

_rowspec = pl.BlockSpec((BR, D), lambda i: (i, 0))
_vecspec = pl.BlockSpec((BR,), lambda i: (i,))
_wspec = pl.BlockSpec((D, D), lambda i: (0, 0))
_sspec = pl.BlockSpec((1, 2), lambda i: (0, 0))
_s1spec = pl.BlockSpec((1, 1), lambda i: (0, 0))
_Sspec = pl.BlockSpec((2, BR, D), lambda i: (0, i, 0))
_dpspec = pl.BlockSpec((NTILES, BR), lambda i: (0, i))


def _tc_xs(res, linWT, scal):
    return pl.pallas_call(
        _tc_xs_body,
        grid=(_GRID,),
        in_specs=[_rowspec, _wspec, _s1spec],
        out_specs=[_rowspec, _vecspec],
        out_shape=[jax.ShapeDtypeStruct((NPAD, D), _f32),
                   jax.ShapeDtypeStruct((NPAD,), _f32)],
    )(res, linWT, scal)


def _tc_comb(mp, a):
    CB = 2048
    return pl.pallas_call(
        _tc_comb_body,
        grid=(NPAD // CB,),
        in_specs=[pl.BlockSpec((NTILES, CB), lambda i: (0, i)),
                  pl.BlockSpec((CB,), lambda i: (i,))],
        out_specs=pl.BlockSpec((CB,), lambda i: (i,)),
        out_shape=jax.ShapeDtypeStruct((NPAD,), _f32),
    )(mp, a)


def _tc_tail(res, xs, a, M, S, dp, th, scal):
    return pl.pallas_call(
        _tc_tail_body,
        grid=(_GRID,),
        in_specs=[_rowspec, _rowspec, _vecspec, _vecspec, _Sspec, _dpspec,
                  _vecspec, _sspec],
        out_specs=_rowspec,
        out_shape=jax.ShapeDtypeStruct((NPAD, D), _f32),
    )(res, xs, a, M, S, dp, th, scal)


@functools.lru_cache(maxsize=None)
def _get_sc_max():
    mesh = plsc.VectorSubcoreMesh(core_axis_name="c", subcore_axis_name="s")
    return functools.partial(
        pl.kernel,
        out_type=jax.ShapeDtypeStruct((NTILES, NPAD), _f32),
        mesh=mesh,
        compiler_params=pltpu.CompilerParams(needs_layout_passes=False),
        scratch_types=[
            pltpu.VMEM((NPAD,), _f32),
            pltpu.VMEM((NPAD,), _f32),
            pltpu.VMEM((EPT,), jnp.int32),
            pltpu.VMEM((EPT,), jnp.int32),
        ],
    )(_sc_max_body)


def _sc_max_body(src_hbm, dst_hbm, a_hbm, out_hbm, a_v, m_v, s_v, d_v):
    cid = lax.axis_index("c")
    sid = lax.axis_index("s")
    wid = sid * 2 + cid
    base = wid * EPT
    pltpu.sync_copy(a_hbm, a_v)
    pltpu.sync_copy(src_hbm.at[pl.ds(base, EPT)], s_v)
    pltpu.sync_copy(dst_hbm.at[pl.ds(base, EPT)], d_v)

    def init_body(i, carry):
        m_v[pl.ds(i * 16, 16)] = jnp.full((16,), NEG, _f32)
        return carry

    lax.fori_loop(0, NPAD // 16, init_body, 0)

    def edge_body(i, carry):
        s = s_v[pl.ds(i * 16, 16)]
        d = d_v[pl.ds(i * 16, 16)]
        g = plsc.load_gather(a_v, [s])
        ok = s != d
        g = jnp.where(ok, g, NEG)
        need = jnp.logical_and(ok, plsc.load_gather(m_v, [d]) < g)

        def wcond(need):
            return jnp.any(need)

        def wbody(need):
            plsc.store_scatter(m_v, [d], g, mask=need)
            cur = plsc.load_gather(m_v, [d])
            return jnp.logical_and(need, cur < g)

        lax.while_loop(wcond, wbody, need)
        return carry

    lax.fori_loop(0, EPT // 16, edge_body, 0)
    pltpu.sync_copy(m_v, out_hbm.at[wid])


@functools.lru_cache(maxsize=None)
def _get_sc_heavy():
    mesh = plsc.VectorSubcoreMesh(core_axis_name="c", subcore_axis_name="s")
    return functools.partial(
        pl.kernel,
        out_type=(jax.ShapeDtypeStruct((2, NPAD, D), _f32),
                  jax.ShapeDtypeStruct((NTILES, NPAD), _f32)),
        mesh=mesh,
        compiler_params=pltpu.CompilerParams(needs_layout_passes=False),
        scratch_types=[
            pltpu.VMEM((NPAD,), _f32),
            pltpu.VMEM((NPAD,), _f32),
            pltpu.VMEM((NPAD,), _f32),
            pltpu.VMEM((ECHUNK,), jnp.int32),
            pltpu.VMEM((ECHUNK,), jnp.int32),
            pltpu.VMEM((16, D), _f32),
            pltpu.VMEM((16, D), _f32),
            pltpu.VMEM_SHARED((NPAD, D), _f32),
            pltpu.SemaphoreType.DMA,
        ],
    )(_sc_heavy_body)


def _sc_heavy_body(src_hbm, dst_hbm, a_hbm, M_hbm, xs_hbm, S_hbm, den_hbm,
                   a_v, m_v, den_v, s_v, d_v, rows_v, srow_v, S_sh, sem):
    cid = lax.axis_index("c")
    sid = lax.axis_index("s")
    wid = sid * 2 + cid
    base = wid * EPT
    pltpu.sync_copy(a_hbm, a_v)
    pltpu.sync_copy(M_hbm, m_v)

    def zden(i, carry):
        den_v[pl.ds(i * 16, 16)] = jnp.zeros((16,), _f32)
        return carry

    lax.fori_loop(0, NPAD // 16, zden, 0)
    for j in range(16):
        for cb in range(D // 16):
            rows_v[j, pl.ds(cb * 16, 16)] = jnp.zeros((16,), _f32)

    def zsh(i, carry):
        pltpu.sync_copy(rows_v, S_sh.at[pl.ds(sid * ROWS_PT + i * 16, 16)])
        return carry

    lax.fori_loop(0, ROWS_PT // 16, zsh, 0)
    plsc.subcore_barrier()

    def chunk_body(k, carry):
        pltpu.sync_copy(src_hbm.at[pl.ds(base + k * ECHUNK, ECHUNK)], s_v)
        pltpu.sync_copy(dst_hbm.at[pl.ds(base + k * ECHUNK, ECHUNK)], d_v)

        def edge_body(i, carry2):
            s = s_v[pl.ds(i * 16, 16)]
            d = d_v[pl.ds(i * 16, 16)]
            g = plsc.load_gather(a_v, [s])
            m = plsc.load_gather(m_v, [d])
            ok = s != d
            ex = jnp.where(ok, jnp.exp(g - m), 0.0)
            plsc.addupdate_scatter(den_v, [d], ex)
            pltpu.async_copy(xs_hbm.at[s], rows_v, sem).wait()
            for j in range(16):
                e = ex[j]
                for cb in range(D // 16):
                    srow_v[j, pl.ds(cb * 16, 16)] = rows_v[j, pl.ds(cb * 16, 16)] * e
            pltpu.sync_copy(srow_v, S_sh.at[d], add=True)
            return carry2

        lax.fori_loop(0, ECHUNK // 16, edge_body, 0)
        return carry

    lax.fori_loop(0, EPT // ECHUNK, chunk_body, 0)
    pltpu.sync_copy(den_v, den_hbm.at[wid])
    plsc.subcore_barrier()
    pltpu.sync_copy(S_sh.at[pl.ds(sid * ROWS_PT, ROWS_PT)],
                    S_hbm.at[cid, pl.ds(sid * ROWS_PT, ROWS_PT)])


def kernel(x, edge_index, W0, b0, lin0, W1, b1, lin1, c0, c1, c2):
    n = x.shape[0]
    src = edge_index[0]
    dst = edge_index[1]
    loops = jnp.arange(n, dtype=src.dtype)
    src_j = jnp.concatenate([src, loops])
    dst_j = jnp.concatenate([dst, loops])
    edge_mask = jnp.concatenate([src != dst, jnp.ones((n,), dtype=jnp.bool_)])

    cin0 = jax.nn.softplus(c0)
    cin1 = jax.nn.softplus(c1)
    xh = jnp.concatenate([jnp.zeros((n, 1), x.dtype), x], axis=1)
    xh = _r_proj(_r_expmap0(_r_proj_tan0(xh), cin0), cin0)
    xh = _r_block0(xh, src_j, dst_j, edge_mask, W0, b0, lin0, cin0, cin1, n)
    res1 = _r_prelude(xh, W1, b1, cin1)

    cin2 = jax.nn.softplus(c2)
    sq1 = 1.0 / jnp.sqrt(cin1[0])
    sq2 = 1.0 / jnp.sqrt(cin2[0])
    s1 = jnp.reshape(sq1, (1, 1))
    s12 = jnp.stack([sq1, sq2]).reshape(1, 2)

    sqrtK1 = jnp.sqrt(1.0 / cin1)
    normu = jnp.minimum(_r_minkowski_norm(res1), MAX_NORM)
    theta1 = jnp.maximum(normu / sqrtK1, MIN_NORM)[:, 0]

    res1_pad = jnp.zeros((NPAD, D), _f32).at[:n].set(res1)
    th_pad = jnp.ones((NPAD,), _f32).at[:n].set(theta1)
    xs1, a1 = _tc_xs(res1_pad, lin1.T, s1)

    sc_max = _get_sc_max()
    sc_heavy = _get_sc_heavy()
    mp = sc_max(src, dst, a1)
    M1 = _tc_comb(mp, a1)
    S, dp = sc_heavy(src, dst, a1, M1, xs1)

    out = _tc_tail(res1_pad, xs1, a1, M1, S, dp, th_pad, s12)
    return out[:n]


# speedup vs baseline: 5.7849x; 3.2725x over previous
import functools
import jax
import jax.numpy as jnp
from jax import lax
from jax.experimental import pallas as pl
from jax.experimental.pallas import tpu as pltpu
from jax.experimental.pallas import tpu_sc as plsc

MIN_NORM = 1e-5
EPS = 1e-7
MAX_NORM = 1e6

N = 10000
E = 320000
D = 128
NPAD = 10240
BR = 256
NTILES = 32
EPT = E // NTILES
ROWS_PT = NPAD // 16
ECHUNK = 2000
NEG = -3.0e38

_f32 = jnp.float32


def _r_cosh(x):
    return jnp.cosh(jnp.clip(x, -15.0, 15.0))


def _r_sinh(x):
    return jnp.sinh(jnp.clip(x, -15.0, 15.0))


def _r_proj(x, c):
    K = 1.0 / c
    y = x[:, 1:]
    y_sqnorm = jnp.sum(y * y, axis=1, keepdims=True)
    head = jnp.sqrt(jnp.maximum(K + y_sqnorm, EPS))
    return jnp.concatenate([head, y], axis=1)


def _r_expmap0(u, c):
    K = 1.0 / c
    sqrtK = jnp.sqrt(K)
    xt = u[:, 1:]
    x_norm = jnp.maximum(jnp.sqrt(jnp.sum(xt * xt, axis=1, keepdims=True)), MIN_NORM)
    theta = x_norm / sqrtK
    head = sqrtK * _r_cosh(theta)
    tail = sqrtK * _r_sinh(theta) * xt / x_norm
    return _r_proj(jnp.concatenate([head * jnp.ones_like(x_norm), tail], axis=1), c)


def _r_logmap0(x, c):
    K = 1.0 / c
    sqrtK = jnp.sqrt(K)
    y = x[:, 1:]
    y_norm = jnp.maximum(jnp.sqrt(jnp.sum(y * y, axis=1, keepdims=True)), MIN_NORM)
    theta = jnp.maximum(x[:, 0:1] / sqrtK, 1.0 + EPS)
    tail = sqrtK * jnp.arccosh(jnp.maximum(theta, 1.0 + EPS)) * y / y_norm
    return jnp.concatenate([jnp.zeros_like(tail[:, :1]), tail], axis=1)


def _r_minkowski_norm(u):
    dot = jnp.sum(u * u, axis=-1, keepdims=True) - 2.0 * u[..., 0:1] * u[..., 0:1]
    return jnp.sqrt(jnp.maximum(dot, EPS))


def _r_expmap(u, x, c):
    K = 1.0 / c
    sqrtK = jnp.sqrt(K)
    normu = jnp.minimum(_r_minkowski_norm(u), MAX_NORM)
    theta = jnp.maximum(normu / sqrtK, MIN_NORM)
    result = _r_cosh(theta) * x + _r_sinh(theta) * u / theta
    return _r_proj(result, c)


def _r_proj_tan(u, x, c):
    ux = jnp.sum(x[:, 1:] * u[:, 1:], axis=1, keepdims=True)
    head = ux / jnp.maximum(x[:, 0:1], EPS)
    return jnp.concatenate([head, u[:, 1:]], axis=1)


def _r_ptransp0(x, u, c):
    K = 1.0 / c
    sqrtK = jnp.sqrt(K)
    x0 = x[:, 0:1]
    y = x[:, 1:]
    y_norm = jnp.maximum(jnp.sqrt(jnp.sum(y * y, axis=1, keepdims=True)), MIN_NORM)
    y_n = y / y_norm
    v = jnp.concatenate([-y_norm, (sqrtK - x0) * y_n], axis=1)
    alpha = jnp.sum(y_n * u[:, 1:], axis=1, keepdims=True) / sqrtK
    res = u - alpha * v
    return _r_proj_tan(res, x, c)


def _r_proj_tan0(u):
    return jnp.concatenate([jnp.zeros_like(u[:, 0:1]), u[:, 1:]], axis=1)


def _r_mobius_add(x, y, c):
    u = _r_logmap0(y, c)
    v = _r_ptransp0(x, u, c)
    return _r_expmap(v, x, c)


def _r_prelude(x, W, b, c_in):
    mv = _r_expmap0(_r_logmap0(x, c_in) @ W, c_in)
    res = _r_proj(mv, c_in)
    bias = _r_proj_tan0(b.reshape(1, -1))
    hyp_bias = _r_proj(_r_expmap0(bias, c_in), c_in)
    return _r_proj(_r_mobius_add(res, hyp_bias, c_in), c_in)


def _r_block0(x, src, dst, edge_mask, W, b, linW, c_in, c_out, n):
    res = _r_prelude(x, W, b, c_in)
    x_tan = _r_logmap0(res, c_in)
    xs = x_tan @ linW.T
    a = jnp.sum(xs, axis=-1)
    ae = jnp.where(edge_mask, a[src] + a[dst], -jnp.inf)
    amax = jax.ops.segment_max(ae, dst, num_segments=n)
    ex = jnp.exp(ae - amax[dst])
    denom = jax.ops.segment_sum(ex, dst, num_segments=n)
    alpha = ex / (denom[dst] + 1e-16)
    agg = jax.ops.segment_sum(alpha[:, None] * xs[src], dst, num_segments=n)
    out = _r_proj(_r_expmap(res, agg, c_in), c_in)
    out = jax.nn.relu(_r_logmap0(out, c_in))
    out = _r_proj_tan0(out)
    return _r_proj(_r_expmap0(out, c_out), c_out)


def _masks(r):
    col = lax.broadcasted_iota(jnp.int32, (r, D), 1)
    is0 = col == 0
    m1 = jnp.where(is0, 0.0, 1.0)
    return is0, m1


def _cosh(x):
    x = jnp.clip(x, -15.0, 15.0)
    return 0.5 * (jnp.exp(x) + jnp.exp(-x))


def _sinh(x):
    x = jnp.clip(x, -15.0, 15.0)
    return 0.5 * (jnp.exp(x) - jnp.exp(-x))


def _head(z, is0):
    return jnp.sum(jnp.where(is0, z, 0.0), axis=1, keepdims=True)


def _proj(z, K, is0, m1):
    y = z * m1
    ysq = jnp.sum(y * y, axis=1, keepdims=True)
    head = jnp.sqrt(jnp.maximum(K + ysq, EPS))
    return jnp.where(is0, head, z)


def _expmap0(u, sqrtK, K, is0, m1):
    xt = u * m1
    xn = jnp.maximum(jnp.sqrt(jnp.sum(xt * xt, axis=1, keepdims=True)), MIN_NORM)
    theta = xn / sqrtK
    head = sqrtK * _cosh(theta)
    tail = sqrtK * _sinh(theta) * xt / xn
    z = jnp.where(is0, head, tail)
    return _proj(z, K, is0, m1)


def _logmap0(z, sqrtK, is0, m1):
    y = z * m1
    yn = jnp.maximum(jnp.sqrt(jnp.sum(y * y, axis=1, keepdims=True)), MIN_NORM)
    x0 = _head(z, is0)
    theta = jnp.maximum(x0 / sqrtK, 1.0 + EPS)
    arc = jnp.log(theta + jnp.sqrt(jnp.maximum((theta - 1.0) * (theta + 1.0), 0.0)))
    return sqrtK * arc * y / yn


def _expmap(u, x, sqrtK, K, is0, m1):
    u0 = _head(u, is0)
    dot = jnp.sum(u * u, axis=1, keepdims=True) - 2.0 * u0 * u0
    normu = jnp.sqrt(jnp.maximum(dot, EPS))
    normu = jnp.minimum(normu, MAX_NORM)
    theta = jnp.maximum(normu / sqrtK, MIN_NORM)
    result = _cosh(theta) * x + _sinh(theta) * u / theta
    return _proj(result, K, is0, m1)


def _tc_xs_body(res_ref, linWT_ref, s_ref, xs_ref, a_ref):
    is0, m1 = _masks(BR)
    sqrtK = s_ref[0, 0]
    x_tan = _logmap0(res_ref[...], sqrtK, is0, m1)
    xs = jnp.dot(x_tan, linWT_ref[...], preferred_element_type=_f32)
    xs_ref[...] = xs
    a_ref[...] = jnp.sum(xs, axis=-1)


def _tc_comb_body(mp_ref, a_ref, out_ref):
    m = jnp.max(mp_ref[...], axis=0)
    out_ref[...] = jnp.maximum(m, a_ref[...])


def _tc_tail_body(res_ref, xs_ref, a_ref, M_ref, S_ref, dp_ref, th_ref, s_ref,
                  out_ref):
    is0, m1 = _masks(BR)
    sqrtK_in = s_ref[0, 0]
    sqrtK_out = s_ref[0, 1]
    K_in = sqrtK_in * sqrtK_in
    K_out = sqrtK_out * sqrtK_out
    xs = xs_ref[...]
    res = res_ref[...]
    ex_self = jnp.exp(a_ref[...] - M_ref[...])[:, None]
    den = (jnp.sum(dp_ref[...], axis=0) + 1e-16)[:, None] + ex_self
    num = S_ref[0] + S_ref[1] + ex_self * xs
    agg = num / den
    theta = th_ref[...][:, None]
    out = _cosh(theta) * agg + _sinh(theta) * res / theta
    out = _proj(out, K_in, is0, m1)
    out = _proj(out, K_in, is0, m1)
    out = jax.nn.relu(_logmap0(out, sqrtK_in, is0, m1)) * m1
    out_ref[...] = _proj(_expmap0(out, sqrtK_out, K_out, is0, m1), K_out, is0, m1)


_GRID = NPAD // BR

_rowspec = pl.BlockSpec((BR, D), lambda i: (i, 0))
_vecspec = pl.BlockSpec((BR,), lambda i: (i,))
_wspec = pl.BlockSpec((D, D), lambda i: (0, 0))
_sspec = pl.BlockSpec((1, 2), lambda i: (0, 0))
_s1spec = pl.BlockSpec((1, 1), lambda i: (0, 0))
_Sspec = pl.BlockSpec((2, BR, D), lambda i: (0, i, 0))
_dpspec = pl.BlockSpec((NTILES, BR), lambda i: (0, i))


def _tc_xs(res, linWT, scal):
    return pl.pallas_call(
        _tc_xs_body,
        grid=(_GRID,),
        in_specs=[_rowspec, _wspec, _s1spec],
        out_specs=[_rowspec, _vecspec],
        out_shape=[jax.ShapeDtypeStruct((NPAD, D), _f32),
                   jax.ShapeDtypeStruct((NPAD,), _f32)],
    )(res, linWT, scal)


def _tc_comb(mp, a):
    CB = 2048
    return pl.pallas_call(
        _tc_comb_body,
        grid=(NPAD // CB,),
        in_specs=[pl.BlockSpec((NTILES, CB), lambda i: (0, i)),
                  pl.BlockSpec((CB,), lambda i: (i,))],
        out_specs=pl.BlockSpec((CB,), lambda i: (i,)),
        out_shape=jax.ShapeDtypeStruct((NPAD,), _f32),
    )(mp, a)


def _tc_tail(res, xs, a, M, S, dp, th, scal):
    return pl.pallas_call(
        _tc_tail_body,
        grid=(_GRID,),
        in_specs=[_rowspec, _rowspec, _vecspec, _vecspec, _Sspec, _dpspec,
                  _vecspec, _sspec],
        out_specs=_rowspec,
        out_shape=jax.ShapeDtypeStruct((NPAD, D), _f32),
    )(res, xs, a, M, S, dp, th, scal)


EJ = E + N
EJP = 330240
EPT2 = EJP // NTILES
ECH2 = 2064


@functools.lru_cache(maxsize=None)
def _get_sc_ae():
    mesh = plsc.VectorSubcoreMesh(core_axis_name="c", subcore_axis_name="s")
    return functools.partial(
        pl.kernel,
        out_type=(jax.ShapeDtypeStruct((EJP,), _f32),
                  jax.ShapeDtypeStruct((NTILES, NPAD), _f32)),
        mesh=mesh,
        compiler_params=pltpu.CompilerParams(needs_layout_passes=False),
        scratch_types=[
            pltpu.VMEM((NPAD,), _f32),
            pltpu.VMEM((NPAD,), _f32),
            pltpu.VMEM((ECH2,), jnp.int32),
            pltpu.VMEM((ECH2,), jnp.int32),
            pltpu.VMEM((ECH2,), _f32),
        ],
    )(_sc_ae_body)


def _sc_ae_body(src_hbm, dst_hbm, a_hbm, ae_hbm, mp_hbm, a_v, m_v, s_v, d_v, o_v):
    cid = lax.axis_index("c")
    sid = lax.axis_index("s")
    wid = sid * 2 + cid
    base = wid * EPT2
    pltpu.sync_copy(a_hbm, a_v)

    def init_body(i, carry):
        m_v[pl.ds(i * 16, 16)] = jnp.full((16,), -jnp.inf, _f32)
        return carry

    lax.fori_loop(0, NPAD // 16, init_body, 0)

    def chunk_body(k, carry):
        cbase = base + k * ECH2
        pltpu.sync_copy(src_hbm.at[pl.ds(cbase, ECH2)], s_v)
        pltpu.sync_copy(dst_hbm.at[pl.ds(cbase, ECH2)], d_v)

        def edge_body(i, carry2):
            s = s_v[pl.ds(i * 16, 16)]
            d = d_v[pl.ds(i * 16, 16)]
            eidx = cbase + i * 16 + lax.iota(jnp.int32, 16)
            ok = jnp.logical_or(
                jnp.logical_and(eidx < E, s != d),
                jnp.logical_and(eidx >= E, eidx < EJ))
            gs = plsc.load_gather(a_v, [s])
            gd = plsc.load_gather(a_v, [d])
            ae = jnp.where(ok, gs + gd, -jnp.inf)
            o_v[pl.ds(i * 16, 16)] = ae
            need = jnp.logical_and(ok, plsc.load_gather(m_v, [d]) < ae)

            def wcond(need):
                return jnp.any(need)

            def wbody(need):
                plsc.store_scatter(m_v, [d], ae, mask=need)
                cur = plsc.load_gather(m_v, [d])
                return jnp.logical_and(need, cur < ae)

            lax.while_loop(wcond, wbody, need)
            return carry2

        lax.fori_loop(0, ECH2 // 16, edge_body, 0)
        pltpu.sync_copy(o_v, ae_hbm.at[pl.ds(cbase, ECH2)])
        return carry

    lax.fori_loop(0, EPT2 // ECH2, chunk_body, 0)
    pltpu.sync_copy(m_v, mp_hbm.at[wid])


@functools.lru_cache(maxsize=None)
def _get_sc_edgesub():
    mesh = plsc.VectorSubcoreMesh(core_axis_name="c", subcore_axis_name="s")
    return functools.partial(
        pl.kernel,
        out_type=jax.ShapeDtypeStruct((EJP,), _f32),
        mesh=mesh,
        compiler_params=pltpu.CompilerParams(needs_layout_passes=False),
        scratch_types=[
            pltpu.VMEM((NPAD,), _f32),
            pltpu.VMEM((ECH2,), jnp.int32),
            pltpu.VMEM((ECH2,), _f32),
            pltpu.VMEM((ECH2,), _f32),
        ],
    )(_sc_edgesub_body)


def _sc_edgesub_body(dst_hbm, ae_hbm, tab_hbm, out_hbm, t_v, d_v, e_v, o_v):
    cid = lax.axis_index("c")
    sid = lax.axis_index("s")
    wid = sid * 2 + cid
    base = wid * EPT2
    pltpu.sync_copy(tab_hbm, t_v)

    def chunk_body(k, carry):
        cbase = base + k * ECH2
        pltpu.sync_copy(dst_hbm.at[pl.ds(cbase, ECH2)], d_v)
        pltpu.sync_copy(ae_hbm.at[pl.ds(cbase, ECH2)], e_v)

        def edge_body(i, carry2):
            d = d_v[pl.ds(i * 16, 16)]
            ae = e_v[pl.ds(i * 16, 16)]
            o_v[pl.ds(i * 16, 16)] = ae - plsc.load_gather(t_v, [d])
            return carry2

        lax.fori_loop(0, ECH2 // 16, edge_body, 0)
        pltpu.sync_copy(o_v, out_hbm.at[pl.ds(cbase, ECH2)])
        return carry

    lax.fori_loop(0, EPT2 // ECH2, chunk_body, 0)


@functools.lru_cache(maxsize=None)
def _get_sc_edgegather():
    mesh = plsc.VectorSubcoreMesh(core_axis_name="c", subcore_axis_name="s")
    return functools.partial(
        pl.kernel,
        out_type=jax.ShapeDtypeStruct((EJP,), _f32),
        mesh=mesh,
        compiler_params=pltpu.CompilerParams(needs_layout_passes=False),
        scratch_types=[
            pltpu.VMEM((NPAD,), _f32),
            pltpu.VMEM((ECH2,), jnp.int32),
            pltpu.VMEM((ECH2,), _f32),
        ],
    )(_sc_edgegather_body)


def _sc_edgegather_body(dst_hbm, tab_hbm, out_hbm, t_v, d_v, o_v):
    cid = lax.axis_index("c")
    sid = lax.axis_index("s")
    wid = sid * 2 + cid
    base = wid * EPT2
    pltpu.sync_copy(tab_hbm, t_v)

    def chunk_body(k, carry):
        cbase = base + k * ECH2
        pltpu.sync_copy(dst_hbm.at[pl.ds(cbase, ECH2)], d_v)

        def edge_body(i, carry2):
            d = d_v[pl.ds(i * 16, 16)]
            o_v[pl.ds(i * 16, 16)] = plsc.load_gather(t_v, [d])
            return carry2

        lax.fori_loop(0, ECH2 // 16, edge_body, 0)
        pltpu.sync_copy(o_v, out_hbm.at[pl.ds(cbase, ECH2)])
        return carry

    lax.fori_loop(0, EPT2 // ECH2, chunk_body, 0)


@functools.lru_cache(maxsize=None)
def _get_sc_upd():
    mesh = plsc.VectorSubcoreMesh(core_axis_name="c", subcore_axis_name="s")
    return functools.partial(
        pl.kernel,
        out_type=jax.ShapeDtypeStruct((EJP, D), _f32),
        mesh=mesh,
        compiler_params=pltpu.CompilerParams(needs_layout_passes=False),
        scratch_types=[
            pltpu.VMEM((ECH2,), jnp.int32),
            pltpu.VMEM((ECH2,), _f32),
            pltpu.VMEM((16, D), _f32),
            pltpu.VMEM((16, D), _f32),
            pltpu.SemaphoreType.DMA,
        ],
    )(_sc_upd_body)


def _sc_upd_body(src_hbm, al_hbm, xs_hbm, out_hbm, s_v, al_v, rows_v, srow_v, sem):
    cid = lax.axis_index("c")
    sid = lax.axis_index("s")
    wid = sid * 2 + cid
    base = wid * EPT2

    def chunk_body(k, carry):
        cbase = base + k * ECH2
        pltpu.sync_copy(src_hbm.at[pl.ds(cbase, ECH2)], s_v)
        pltpu.sync_copy(al_hbm.at[pl.ds(cbase, ECH2)], al_v)

        def edge_body(i, carry2):
            s = s_v[pl.ds(i * 16, 16)]
            al = al_v[pl.ds(i * 16, 16)]
            pltpu.async_copy(xs_hbm.at[s], rows_v, sem).wait()
            for j in range(16):
                e = al[j]
                for cb in range(D // 16):
                    srow_v[j, pl.ds(cb * 16, 16)] = rows_v[j, pl.ds(cb * 16, 16)] * e
            pltpu.sync_copy(srow_v, out_hbm.at[pl.ds(cbase + i * 16, 16)])
            return carry2

        lax.fori_loop(0, ECH2 // 16, edge_body, 0)
        return carry

    lax.fori_loop(0, EPT2 // ECH2, chunk_body, 0)


def _tc_comb2_body(mp_ref, out_ref):
    out_ref[...] = jnp.max(mp_ref[...], axis=0)


def _tc_comb2(mp):
    CB = 2048
    return pl.pallas_call(
        _tc_comb2_body,
        grid=(NPAD // CB,),
        in_specs=[pl.BlockSpec((NTILES, CB), lambda i: (0, i))],
        out_specs=pl.BlockSpec((CB,), lambda i: (i,)),
        out_shape=jax.ShapeDtypeStruct((NPAD,), _f32),
    )(mp)


@functools.lru_cache(maxsize=None)
def _get_sc_max():
    mesh = plsc.VectorSubcoreMesh(core_axis_name="c", subcore_axis_name="s")
    return functools.partial(
        pl.kernel,
        out_type=jax.ShapeDtypeStruct((NTILES, NPAD), _f32),
        mesh=mesh,
        compiler_params=pltpu.CompilerParams(needs_layout_passes=False),
        scratch_types=[
            pltpu.VMEM((NPAD,), _f32),
            pltpu.VMEM((NPAD,), _f32),
            pltpu.VMEM((EPT,), jnp.int32),
            pltpu.VMEM((EPT,), jnp.int32),
        ],
    )(_sc_max_body)


def _sc_max_body(src_hbm, dst_hbm, a_hbm, out_hbm, a_v, m_v, s_v, d_v):
    cid = lax.axis_index("c")
    sid = lax.axis_index("s")
    wid = sid * 2 + cid
    base = wid * EPT
    pltpu.sync_copy(a_hbm, a_v)
    pltpu.sync_copy(src_hbm.at[pl.ds(base, EPT)], s_v)
    pltpu.sync_copy(dst_hbm.at[pl.ds(base, EPT)], d_v)

    def init_body(i, carry):
        m_v[pl.ds(i * 16, 16)] = jnp.full((16,), NEG, _f32)
        return carry

    lax.fori_loop(0, NPAD // 16, init_body, 0)

    def edge_body(i, carry):
        s = s_v[pl.ds(i * 16, 16)]
        d = d_v[pl.ds(i * 16, 16)]
        g = plsc.load_gather(a_v, [s])
        ok = s != d
        g = jnp.where(ok, g, NEG)
        need = jnp.logical_and(ok, plsc.load_gather(m_v, [d]) < g)

        def wcond(need):
            return jnp.any(need)

        def wbody(need):
            plsc.store_scatter(m_v, [d], g, mask=need)
            cur = plsc.load_gather(m_v, [d])
            return jnp.logical_and(need, cur < g)

        lax.while_loop(wcond, wbody, need)
        return carry

    lax.fori_loop(0, EPT // 16, edge_body, 0)
    pltpu.sync_copy(m_v, out_hbm.at[wid])


@functools.lru_cache(maxsize=None)
def _get_sc_heavy():
    mesh = plsc.VectorSubcoreMesh(core_axis_name="c", subcore_axis_name="s")
    return functools.partial(
        pl.kernel,
        out_type=(jax.ShapeDtypeStruct((2, NPAD, D), _f32),
                  jax.ShapeDtypeStruct((NTILES, NPAD), _f32)),
        mesh=mesh,
        compiler_params=pltpu.CompilerParams(needs_layout_passes=False),
        scratch_types=[
            pltpu.VMEM((NPAD,), _f32),
            pltpu.VMEM((NPAD,), _f32),
            pltpu.VMEM((NPAD,), _f32),
            pltpu.VMEM((ECHUNK,), jnp.int32),
            pltpu.VMEM((ECHUNK,), jnp.int32),
            pltpu.VMEM((16, D), _f32),
            pltpu.VMEM((16, D), _f32),
            pltpu.VMEM_SHARED((NPAD, D), _f32),
            pltpu.SemaphoreType.DMA,
        ],
    )(_sc_heavy_body)


def _sc_heavy_body(src_hbm, dst_hbm, a_hbm, M_hbm, xs_hbm, S_hbm, den_hbm,
                   a_v, m_v, den_v, s_v, d_v, rows_v, srow_v, S_sh, sem):
    cid = lax.axis_index("c")
    sid = lax.axis_index("s")
    wid = sid * 2 + cid
    base = wid * EPT
    pltpu.sync_copy(a_hbm, a_v)
    pltpu.sync_copy(M_hbm, m_v)

    def zden(i, carry):
        den_v[pl.ds(i * 16, 16)] = jnp.zeros((16,), _f32)
        return carry

    lax.fori_loop(0, NPAD // 16, zden, 0)
    for j in range(16):
        for cb in range(D // 16):
            rows_v[j, pl.ds(cb * 16, 16)] = jnp.zeros((16,), _f32)

    def zsh(i, carry):
        pltpu.sync_copy(rows_v, S_sh.at[pl.ds(sid * ROWS_PT + i * 16, 16)])
        return carry

    lax.fori_loop(0, ROWS_PT // 16, zsh, 0)
    plsc.subcore_barrier()

    def chunk_body(k, carry):
        pltpu.sync_copy(src_hbm.at[pl.ds(base + k * ECHUNK, ECHUNK)], s_v)
        pltpu.sync_copy(dst_hbm.at[pl.ds(base + k * ECHUNK, ECHUNK)], d_v)

        def edge_body(i, carry2):
            s = s_v[pl.ds(i * 16, 16)]
            d = d_v[pl.ds(i * 16, 16)]
            g = plsc.load_gather(a_v, [s])
            m = plsc.load_gather(m_v, [d])
            ok = s != d
            ex = jnp.where(ok, jnp.exp(g - m), 0.0)
            plsc.addupdate_scatter(den_v, [d], ex)
            pltpu.async_copy(xs_hbm.at[s], rows_v, sem).wait()
            for j in range(16):
                e = ex[j]
                for cb in range(D // 16):
                    srow_v[j, pl.ds(cb * 16, 16)] = rows_v[j, pl.ds(cb * 16, 16)] * e
            pltpu.sync_copy(srow_v, S_sh.at[d], add=True)
            return carry2

        lax.fori_loop(0, ECHUNK // 16, edge_body, 0)
        return carry

    lax.fori_loop(0, EPT // ECHUNK, chunk_body, 0)
    pltpu.sync_copy(den_v, den_hbm.at[wid])
    plsc.subcore_barrier()
    pltpu.sync_copy(S_sh.at[pl.ds(sid * ROWS_PT, ROWS_PT)],
                    S_hbm.at[cid, pl.ds(sid * ROWS_PT, ROWS_PT)])


def kernel(x, edge_index, W0, b0, lin0, W1, b1, lin1, c0, c1, c2):
    n = x.shape[0]
    src = edge_index[0]
    dst = edge_index[1]
    loops = jnp.arange(n, dtype=src.dtype)
    src_j = jnp.concatenate([src, loops])
    dst_j = jnp.concatenate([dst, loops])
    edge_mask = jnp.concatenate([src != dst, jnp.ones((n,), dtype=jnp.bool_)])

    cin0 = jax.nn.softplus(c0)
    cin1 = jax.nn.softplus(c1)
    xh = jnp.concatenate([jnp.zeros((n, 1), x.dtype), x], axis=1)
    xh = _r_proj(_r_expmap0(_r_proj_tan0(xh), cin0), cin0)

    res0 = _r_prelude(xh, W0, b0, cin0)
    x_tan0 = _r_logmap0(res0, cin0)
    xs0 = x_tan0 @ lin0.T
    a0 = jnp.sum(xs0, axis=-1)

    a0_pad = jnp.zeros((NPAD,), _f32).at[:n].set(a0)
    srcp = jnp.concatenate([src, loops, jnp.zeros((EJP - EJ,), src.dtype)])
    dstp = jnp.concatenate([dst, loops, jnp.zeros((EJP - EJ,), dst.dtype)])

    ae_pad, mp0 = _get_sc_ae()(srcp, dstp, a0_pad)
    amax0 = _tc_comb2(mp0)
    sub0 = _get_sc_edgesub()(dstp, ae_pad, amax0)
    ex0 = jnp.exp(sub0[:EJ])
    denom0 = jax.ops.segment_sum(ex0, dst_j, num_segments=n)
    den_pad = jnp.zeros((NPAD,), _f32).at[:n].set(denom0)
    deng0 = _get_sc_edgegather()(dstp, den_pad)
    alpha0 = ex0 / (deng0[:EJ] + 1e-16)
    alpha0_pad = jnp.concatenate([alpha0, jnp.zeros((EJP - EJ,), _f32)])
    xs0_pad = jnp.zeros((NPAD, D), _f32).at[:n].set(xs0)
    upd0 = _get_sc_upd()(srcp, alpha0_pad, xs0_pad)
    agg0 = jax.ops.segment_sum(upd0[:EJ], dst_j, num_segments=n)

    out0 = _r_proj(_r_expmap(res0, agg0, cin0), cin0)
    out0 = jax.nn.relu(_r_logmap0(out0, cin0))
    out0 = _r_proj_tan0(out0)
    xh = _r_proj(_r_expmap0(out0, cin1), cin1)
    res1 = _r_prelude(xh, W1, b1, cin1)

    cin2 = jax.nn.softplus(c2)
    sq1 = 1.0 / jnp.sqrt(cin1[0])
    sq2 = 1.0 / jnp.sqrt(cin2[0])
    s1 = jnp.reshape(sq1, (1, 1))
    s12 = jnp.stack([sq1, sq2]).reshape(1, 2)

    sqrtK1 = jnp.sqrt(1.0 / cin1)
    normu = jnp.minimum(_r_minkowski_norm(res1), MAX_NORM)
    theta1 = jnp.maximum(normu / sqrtK1, MIN_NORM)[:, 0]

    res1_pad = jnp.zeros((NPAD, D), _f32).at[:n].set(res1)
    th_pad = jnp.ones((NPAD,), _f32).at[:n].set(theta1)
    xs1, a1 = _tc_xs(res1_pad, lin1.T, s1)

    sc_max = _get_sc_max()
    sc_heavy = _get_sc_heavy()
    mp = sc_max(src, dst, a1)
    M1 = _tc_comb(mp, a1)
    S, dp = sc_heavy(src, dst, a1, M1, xs1)

    out = _tc_tail(res1_pad, xs1, a1, M1, S, dp, th_pad, s12)
    return out[:n]


# speedup vs baseline: 6.1790x; 1.0681x over previous
import functools
import jax
import jax.numpy as jnp
from jax import lax
from jax.experimental import pallas as pl
from jax.experimental.pallas import tpu as pltpu
from jax.experimental.pallas import tpu_sc as plsc

MIN_NORM = 1e-5
EPS = 1e-7
MAX_NORM = 1e6

N = 10000
E = 320000
D = 128
NPAD = 10240
BR = 256
NTILES = 32
EPT = E // NTILES
ROWS_PT = NPAD // 16
ECHUNK = 2000
NEG = -3.0e38

_f32 = jnp.float32


def _r_cosh(x):
    return jnp.cosh(jnp.clip(x, -15.0, 15.0))


def _r_sinh(x):
    return jnp.sinh(jnp.clip(x, -15.0, 15.0))


def _r_proj(x, c):
    K = 1.0 / c
    y = x[:, 1:]
    y_sqnorm = jnp.sum(y * y, axis=1, keepdims=True)
    head = jnp.sqrt(jnp.maximum(K + y_sqnorm, EPS))
    return jnp.concatenate([head, y], axis=1)


def _r_expmap0(u, c):
    K = 1.0 / c
    sqrtK = jnp.sqrt(K)
    xt = u[:, 1:]
    x_norm = jnp.maximum(jnp.sqrt(jnp.sum(xt * xt, axis=1, keepdims=True)), MIN_NORM)
    theta = x_norm / sqrtK
    head = sqrtK * _r_cosh(theta)
    tail = sqrtK * _r_sinh(theta) * xt / x_norm
    return _r_proj(jnp.concatenate([head * jnp.ones_like(x_norm), tail], axis=1), c)


def _r_logmap0(x, c):
    K = 1.0 / c
    sqrtK = jnp.sqrt(K)
    y = x[:, 1:]
    y_norm = jnp.maximum(jnp.sqrt(jnp.sum(y * y, axis=1, keepdims=True)), MIN_NORM)
    theta = jnp.maximum(x[:, 0:1] / sqrtK, 1.0 + EPS)
    tail = sqrtK * jnp.arccosh(jnp.maximum(theta, 1.0 + EPS)) * y / y_norm
    return jnp.concatenate([jnp.zeros_like(tail[:, :1]), tail], axis=1)


def _r_minkowski_norm(u):
    dot = jnp.sum(u * u, axis=-1, keepdims=True) - 2.0 * u[..., 0:1] * u[..., 0:1]
    return jnp.sqrt(jnp.maximum(dot, EPS))


def _r_expmap(u, x, c):
    K = 1.0 / c
    sqrtK = jnp.sqrt(K)
    normu = jnp.minimum(_r_minkowski_norm(u), MAX_NORM)
    theta = jnp.maximum(normu / sqrtK, MIN_NORM)
    result = _r_cosh(theta) * x + _r_sinh(theta) * u / theta
    return _r_proj(result, c)


def _r_proj_tan(u, x, c):
    ux = jnp.sum(x[:, 1:] * u[:, 1:], axis=1, keepdims=True)
    head = ux / jnp.maximum(x[:, 0:1], EPS)
    return jnp.concatenate([head, u[:, 1:]], axis=1)


def _r_ptransp0(x, u, c):
    K = 1.0 / c
    sqrtK = jnp.sqrt(K)
    x0 = x[:, 0:1]
    y = x[:, 1:]
    y_norm = jnp.maximum(jnp.sqrt(jnp.sum(y * y, axis=1, keepdims=True)), MIN_NORM)
    y_n = y / y_norm
    v = jnp.concatenate([-y_norm, (sqrtK - x0) * y_n], axis=1)
    alpha = jnp.sum(y_n * u[:, 1:], axis=1, keepdims=True) / sqrtK
    res = u - alpha * v
    return _r_proj_tan(res, x, c)


def _r_proj_tan0(u):
    return jnp.concatenate([jnp.zeros_like(u[:, 0:1]), u[:, 1:]], axis=1)


def _r_mobius_add(x, y, c):
    u = _r_logmap0(y, c)
    v = _r_ptransp0(x, u, c)
    return _r_expmap(v, x, c)


def _r_prelude(x, W, b, c_in):
    mv = _r_expmap0(_r_logmap0(x, c_in) @ W, c_in)
    res = _r_proj(mv, c_in)
    bias = _r_proj_tan0(b.reshape(1, -1))
    hyp_bias = _r_proj(_r_expmap0(bias, c_in), c_in)
    return _r_proj(_r_mobius_add(res, hyp_bias, c_in), c_in)


def _r_block0(x, src, dst, edge_mask, W, b, linW, c_in, c_out, n):
    res = _r_prelude(x, W, b, c_in)
    x_tan = _r_logmap0(res, c_in)
    xs = x_tan @ linW.T
    a = jnp.sum(xs, axis=-1)
    ae = jnp.where(edge_mask, a[src] + a[dst], -jnp.inf)
    amax = jax.ops.segment_max(ae, dst, num_segments=n)
    ex = jnp.exp(ae - amax[dst])
    denom = jax.ops.segment_sum(ex, dst, num_segments=n)
    alpha = ex / (denom[dst] + 1e-16)
    agg = jax.ops.segment_sum(alpha[:, None] * xs[src], dst, num_segments=n)
    out = _r_proj(_r_expmap(res, agg, c_in), c_in)
    out = jax.nn.relu(_r_logmap0(out, c_in))
    out = _r_proj_tan0(out)
    return _r_proj(_r_expmap0(out, c_out), c_out)


def _masks(r):
    col = lax.broadcasted_iota(jnp.int32, (r, D), 1)
    is0 = col == 0
    m1 = jnp.where(is0, 0.0, 1.0)
    return is0, m1


def _cosh(x):
    x = jnp.clip(x, -15.0, 15.0)
    return 0.5 * (jnp.exp(x) + jnp.exp(-x))


def _sinh(x):
    x = jnp.clip(x, -15.0, 15.0)
    return 0.5 * (jnp.exp(x) - jnp.exp(-x))


def _head(z, is0):
    return jnp.sum(jnp.where(is0, z, 0.0), axis=1, keepdims=True)


def _proj(z, K, is0, m1):
    y = z * m1
    ysq = jnp.sum(y * y, axis=1, keepdims=True)
    head = jnp.sqrt(jnp.maximum(K + ysq, EPS))
    return jnp.where(is0, head, z)


def _expmap0(u, sqrtK, K, is0, m1):
    xt = u * m1
    xn = jnp.maximum(jnp.sqrt(jnp.sum(xt * xt, axis=1, keepdims=True)), MIN_NORM)
    theta = xn / sqrtK
    head = sqrtK * _cosh(theta)
    tail = sqrtK * _sinh(theta) * xt / xn
    z = jnp.where(is0, head, tail)
    return _proj(z, K, is0, m1)


def _logmap0(z, sqrtK, is0, m1):
    y = z * m1
    yn = jnp.maximum(jnp.sqrt(jnp.sum(y * y, axis=1, keepdims=True)), MIN_NORM)
    x0 = _head(z, is0)
    theta = jnp.maximum(x0 / sqrtK, 1.0 + EPS)
    arc = jnp.log(theta + jnp.sqrt(jnp.maximum((theta - 1.0) * (theta + 1.0), 0.0)))
    return sqrtK * arc * y / yn


def _expmap(u, x, sqrtK, K, is0, m1):
    u0 = _head(u, is0)
    dot = jnp.sum(u * u, axis=1, keepdims=True) - 2.0 * u0 * u0
    normu = jnp.sqrt(jnp.maximum(dot, EPS))
    normu = jnp.minimum(normu, MAX_NORM)
    theta = jnp.maximum(normu / sqrtK, MIN_NORM)
    result = _cosh(theta) * x + _sinh(theta) * u / theta
    return _proj(result, K, is0, m1)


def _tc_xs_body(res_ref, linWT_ref, s_ref, xs_ref, a_ref):
    is0, m1 = _masks(BR)
    sqrtK = s_ref[0, 0]
    x_tan = _logmap0(res_ref[...], sqrtK, is0, m1)
    xs = jnp.dot(x_tan, linWT_ref[...], preferred_element_type=_f32)
    xs_ref[...] = xs
    a_ref[...] = jnp.sum(xs, axis=-1)


def _tc_comb_body(mp_ref, a_ref, out_ref):
    m = jnp.max(mp_ref[...], axis=0)
    out_ref[...] = jnp.maximum(m, a_ref[...])


def _tc_tail_body(res_ref, xs_ref, a_ref, M_ref, S_ref, dp_ref, th_ref, s_ref,
                  out_ref):
    is0, m1 = _masks(BR)
    sqrtK_in = s_ref[0, 0]
    sqrtK_out = s_ref[0, 1]
    K_in = sqrtK_in * sqrtK_in
    K_out = sqrtK_out * sqrtK_out
    xs = xs_ref[...]
    res = res_ref[...]
    ex_self = jnp.exp(a_ref[...] - M_ref[...])[:, None]
    den = (jnp.sum(dp_ref[...], axis=0) + 1e-16)[:, None] + ex_self
    num = S_ref[0] + S_ref[1] + ex_self * xs
    agg = num / den
    theta = th_ref[...][:, None]
    out = _cosh(theta) * agg + _sinh(theta) * res / theta
    out = _proj(out, K_in, is0, m1)
    out = _proj(out, K_in, is0, m1)
    out = jax.nn.relu(_logmap0(out, sqrtK_in, is0, m1)) * m1
    out_ref[...] = _proj(_expmap0(out, sqrtK_out, K_out, is0, m1), K_out, is0, m1)


_GRID = NPAD // BR

_rowspec = pl.BlockSpec((BR, D), lambda i: (i, 0))
_vecspec = pl.BlockSpec((BR,), lambda i: (i,))
_wspec = pl.BlockSpec((D, D), lambda i: (0, 0))
_sspec = pl.BlockSpec((1, 2), lambda i: (0, 0))
_s1spec = pl.BlockSpec((1, 1), lambda i: (0, 0))
_Sspec = pl.BlockSpec((2, BR, D), lambda i: (0, i, 0))
_dpspec = pl.BlockSpec((NTILES, BR), lambda i: (0, i))


def _tc_xs(res, linWT, scal):
    return pl.pallas_call(
        _tc_xs_body,
        grid=(_GRID,),
        in_specs=[_rowspec, _wspec, _s1spec],
        out_specs=[_rowspec, _vecspec],
        out_shape=[jax.ShapeDtypeStruct((NPAD, D), _f32),
                   jax.ShapeDtypeStruct((NPAD,), _f32)],
    )(res, linWT, scal)


def _tc_comb(mp, a):
    CB = 2048
    return pl.pallas_call(
        _tc_comb_body,
        grid=(NPAD // CB,),
        in_specs=[pl.BlockSpec((NTILES, CB), lambda i: (0, i)),
                  pl.BlockSpec((CB,), lambda i: (i,))],
        out_specs=pl.BlockSpec((CB,), lambda i: (i,)),
        out_shape=jax.ShapeDtypeStruct((NPAD,), _f32),
    )(mp, a)


def _tc_tail(res, xs, a, M, S, dp, th, scal):
    return pl.pallas_call(
        _tc_tail_body,
        grid=(_GRID,),
        in_specs=[_rowspec, _rowspec, _vecspec, _vecspec, _Sspec, _dpspec,
                  _vecspec, _sspec],
        out_specs=_rowspec,
        out_shape=jax.ShapeDtypeStruct((NPAD, D), _f32),
    )(res, xs, a, M, S, dp, th, scal)


EJ = E + N
EJP = 330240
EPT2 = EJP // NTILES
ECH2 = 2064
UB = 48


@functools.lru_cache(maxsize=None)
def _get_sc_ae():
    mesh = plsc.VectorSubcoreMesh(core_axis_name="c", subcore_axis_name="s")
    return functools.partial(
        pl.kernel,
        out_type=(jax.ShapeDtypeStruct((EJP,), _f32),
                  jax.ShapeDtypeStruct((NTILES, NPAD), _f32)),
        mesh=mesh,
        compiler_params=pltpu.CompilerParams(needs_layout_passes=False),
        scratch_types=[
            pltpu.VMEM((NPAD,), _f32),
            pltpu.VMEM((NPAD,), _f32),
            pltpu.VMEM((ECH2,), jnp.int32),
            pltpu.VMEM((ECH2,), jnp.int32),
            pltpu.VMEM((ECH2,), _f32),
        ],
    )(_sc_ae_body)


def _sc_ae_body(src_hbm, dst_hbm, a_hbm, ae_hbm, mp_hbm, a_v, m_v, s_v, d_v, o_v):
    cid = lax.axis_index("c")
    sid = lax.axis_index("s")
    wid = sid * 2 + cid
    base = wid * EPT2
    pltpu.sync_copy(a_hbm, a_v)

    def init_body(i, carry):
        m_v[pl.ds(i * 16, 16)] = jnp.full((16,), -jnp.inf, _f32)
        return carry

    lax.fori_loop(0, NPAD // 16, init_body, 0)

    def chunk_body(k, carry):
        cbase = base + k * ECH2
        pltpu.sync_copy(src_hbm.at[pl.ds(cbase, ECH2)], s_v)
        pltpu.sync_copy(dst_hbm.at[pl.ds(cbase, ECH2)], d_v)

        def edge_body(i, carry2):
            s = s_v[pl.ds(i * 16, 16)]
            d = d_v[pl.ds(i * 16, 16)]
            eidx = cbase + i * 16 + lax.iota(jnp.int32, 16)
            ok = jnp.logical_or(
                jnp.logical_and(eidx < E, s != d),
                jnp.logical_and(eidx >= E, eidx < EJ))
            gs = plsc.load_gather(a_v, [s])
            gd = plsc.load_gather(a_v, [d])
            ae = jnp.where(ok, gs + gd, -jnp.inf)
            o_v[pl.ds(i * 16, 16)] = ae
            need = jnp.logical_and(ok, plsc.load_gather(m_v, [d]) < ae)

            def wcond(need):
                return jnp.any(need)

            def wbody(need):
                plsc.store_scatter(m_v, [d], ae, mask=need)
                cur = plsc.load_gather(m_v, [d])
                return jnp.logical_and(need, cur < ae)

            lax.while_loop(wcond, wbody, need)
            return carry2

        lax.fori_loop(0, ECH2 // 16, edge_body, 0)
        pltpu.sync_copy(o_v, ae_hbm.at[pl.ds(cbase, ECH2)])
        return carry

    lax.fori_loop(0, EPT2 // ECH2, chunk_body, 0)
    pltpu.sync_copy(m_v, mp_hbm.at[wid])


@functools.lru_cache(maxsize=None)
def _get_sc_edgesub():
    mesh = plsc.VectorSubcoreMesh(core_axis_name="c", subcore_axis_name="s")
    return functools.partial(
        pl.kernel,
        out_type=jax.ShapeDtypeStruct((EJP,), _f32),
        mesh=mesh,
        compiler_params=pltpu.CompilerParams(needs_layout_passes=False),
        scratch_types=[
            pltpu.VMEM((NPAD,), _f32),
            pltpu.VMEM((ECH2,), jnp.int32),
            pltpu.VMEM((ECH2,), _f32),
            pltpu.VMEM((ECH2,), _f32),
        ],
    )(_sc_edgesub_body)


def _sc_edgesub_body(dst_hbm, ae_hbm, tab_hbm, out_hbm, t_v, d_v, e_v, o_v):
    cid = lax.axis_index("c")
    sid = lax.axis_index("s")
    wid = sid * 2 + cid
    base = wid * EPT2
    pltpu.sync_copy(tab_hbm, t_v)

    def chunk_body(k, carry):
        cbase = base + k * ECH2
        pltpu.sync_copy(dst_hbm.at[pl.ds(cbase, ECH2)], d_v)
        pltpu.sync_copy(ae_hbm.at[pl.ds(cbase, ECH2)], e_v)

        def edge_body(i, carry2):
            d = d_v[pl.ds(i * 16, 16)]
            ae = e_v[pl.ds(i * 16, 16)]
            o_v[pl.ds(i * 16, 16)] = ae - plsc.load_gather(t_v, [d])
            return carry2

        lax.fori_loop(0, ECH2 // 16, edge_body, 0)
        pltpu.sync_copy(o_v, out_hbm.at[pl.ds(cbase, ECH2)])
        return carry

    lax.fori_loop(0, EPT2 // ECH2, chunk_body, 0)


@functools.lru_cache(maxsize=None)
def _get_sc_edgegather():
    mesh = plsc.VectorSubcoreMesh(core_axis_name="c", subcore_axis_name="s")
    return functools.partial(
        pl.kernel,
        out_type=jax.ShapeDtypeStruct((EJP,), _f32),
        mesh=mesh,
        compiler_params=pltpu.CompilerParams(needs_layout_passes=False),
        scratch_types=[
            pltpu.VMEM((NPAD,), _f32),
            pltpu.VMEM((ECH2,), jnp.int32),
            pltpu.VMEM((ECH2,), _f32),
        ],
    )(_sc_edgegather_body)


def _sc_edgegather_body(dst_hbm, tab_hbm, out_hbm, t_v, d_v, o_v):
    cid = lax.axis_index("c")
    sid = lax.axis_index("s")
    wid = sid * 2 + cid
    base = wid * EPT2
    pltpu.sync_copy(tab_hbm, t_v)

    def chunk_body(k, carry):
        cbase = base + k * ECH2
        pltpu.sync_copy(dst_hbm.at[pl.ds(cbase, ECH2)], d_v)

        def edge_body(i, carry2):
            d = d_v[pl.ds(i * 16, 16)]
            o_v[pl.ds(i * 16, 16)] = plsc.load_gather(t_v, [d])
            return carry2

        lax.fori_loop(0, ECH2 // 16, edge_body, 0)
        pltpu.sync_copy(o_v, out_hbm.at[pl.ds(cbase, ECH2)])
        return carry

    lax.fori_loop(0, EPT2 // ECH2, chunk_body, 0)


@functools.lru_cache(maxsize=None)
def _get_sc_upd():
    mesh = plsc.VectorSubcoreMesh(core_axis_name="c", subcore_axis_name="s")
    return functools.partial(
        pl.kernel,
        out_type=jax.ShapeDtypeStruct((EJP, D), _f32),
        mesh=mesh,
        compiler_params=pltpu.CompilerParams(needs_layout_passes=False),
        scratch_types=[
            pltpu.VMEM((ECH2,), jnp.int32),
            pltpu.VMEM((ECH2,), _f32),
            pltpu.VMEM((UB, D), _f32),
            pltpu.VMEM((UB, D), _f32),
            pltpu.SemaphoreType.DMA,
        ],
    )(_sc_upd_body)


def _sc_upd_body(src_hbm, al_hbm, xs_hbm, out_hbm, s_v, al_v, rows_v, srow_v, sem):
    cid = lax.axis_index("c")
    sid = lax.axis_index("s")
    wid = sid * 2 + cid
    base = wid * EPT2

    def chunk_body(k, carry):
        cbase = base + k * ECH2
        pltpu.sync_copy(src_hbm.at[pl.ds(cbase, ECH2)], s_v)
        pltpu.sync_copy(al_hbm.at[pl.ds(cbase, ECH2)], al_v)

        def edge_body(i, carry2):
            pltpu.async_copy(
                xs_hbm.at[s_v.at[pl.ds(i * UB, UB)]], rows_v, sem).wait()
            for jj in range(UB // 16):
                al = al_v[pl.ds(i * UB + jj * 16, 16)]
                for j in range(16):
                    e = al[j]
                    r = jj * 16 + j
                    for cb in range(D // 16):
                        srow_v[r, pl.ds(cb * 16, 16)] = (
                            rows_v[r, pl.ds(cb * 16, 16)] * e)
            pltpu.sync_copy(srow_v, out_hbm.at[pl.ds(cbase + i * UB, UB)])
            return carry2

        lax.fori_loop(0, ECH2 // UB, edge_body, 0)
        return carry

    lax.fori_loop(0, EPT2 // ECH2, chunk_body, 0)


def _tc_comb2_body(mp_ref, out_ref):
    out_ref[...] = jnp.max(mp_ref[...], axis=0)


def _tc_comb2(mp):
    CB = 2048
    return pl.pallas_call(
        _tc_comb2_body,
        grid=(NPAD // CB,),
        in_specs=[pl.BlockSpec((NTILES, CB), lambda i: (0, i))],
        out_specs=pl.BlockSpec((CB,), lambda i: (i,)),
        out_shape=jax.ShapeDtypeStruct((NPAD,), _f32),
    )(mp)


@functools.lru_cache(maxsize=None)
def _get_sc_max():
    mesh = plsc.VectorSubcoreMesh(core_axis_name="c", subcore_axis_name="s")
    return functools.partial(
        pl.kernel,
        out_type=jax.ShapeDtypeStruct((NTILES, NPAD), _f32),
        mesh=mesh,
        compiler_params=pltpu.CompilerParams(needs_layout_passes=False),
        scratch_types=[
            pltpu.VMEM((NPAD,), _f32),
            pltpu.VMEM((NPAD,), _f32),
            pltpu.VMEM((EPT,), jnp.int32),
            pltpu.VMEM((EPT,), jnp.int32),
        ],
    )(_sc_max_body)


def _sc_max_body(src_hbm, dst_hbm, a_hbm, out_hbm, a_v, m_v, s_v, d_v):
    cid = lax.axis_index("c")
    sid = lax.axis_index("s")
    wid = sid * 2 + cid
    base = wid * EPT
    pltpu.sync_copy(a_hbm, a_v)
    pltpu.sync_copy(src_hbm.at[pl.ds(base, EPT)], s_v)
    pltpu.sync_copy(dst_hbm.at[pl.ds(base, EPT)], d_v)

    def init_body(i, carry):
        m_v[pl.ds(i * 16, 16)] = jnp.full((16,), NEG, _f32)
        return carry

    lax.fori_loop(0, NPAD // 16, init_body, 0)

    def edge_body(i, carry):
        s = s_v[pl.ds(i * 16, 16)]
        d = d_v[pl.ds(i * 16, 16)]
        g = plsc.load_gather(a_v, [s])
        ok = s != d
        g = jnp.where(ok, g, NEG)
        need = jnp.logical_and(ok, plsc.load_gather(m_v, [d]) < g)

        def wcond(need):
            return jnp.any(need)

        def wbody(need):
            plsc.store_scatter(m_v, [d], g, mask=need)
            cur = plsc.load_gather(m_v, [d])
            return jnp.logical_and(need, cur < g)

        lax.while_loop(wcond, wbody, need)
        return carry

    lax.fori_loop(0, EPT // 16, edge_body, 0)
    pltpu.sync_copy(m_v, out_hbm.at[wid])


@functools.lru_cache(maxsize=None)
def _get_sc_heavy():
    mesh = plsc.VectorSubcoreMesh(core_axis_name="c", subcore_axis_name="s")
    return functools.partial(
        pl.kernel,
        out_type=(jax.ShapeDtypeStruct((2, NPAD, D), _f32),
                  jax.ShapeDtypeStruct((NTILES, NPAD), _f32)),
        mesh=mesh,
        compiler_params=pltpu.CompilerParams(needs_layout_passes=False),
        scratch_types=[
            pltpu.VMEM((NPAD,), _f32),
            pltpu.VMEM((NPAD,), _f32),
            pltpu.VMEM((NPAD,), _f32),
            pltpu.VMEM((ECHUNK,), jnp.int32),
            pltpu.VMEM((ECHUNK,), jnp.int32),
            pltpu.VMEM((16, D), _f32),
            pltpu.VMEM((16, D), _f32),
            pltpu.VMEM_SHARED((NPAD, D), _f32),
            pltpu.SemaphoreType.DMA,
        ],
    )(_sc_heavy_body)


def _sc_heavy_body(src_hbm, dst_hbm, a_hbm, M_hbm, xs_hbm, S_hbm, den_hbm,
                   a_v, m_v, den_v, s_v, d_v, rows_v, srow_v, S_sh, sem):
    cid = lax.axis_index("c")
    sid = lax.axis_index("s")
    wid = sid * 2 + cid
    base = wid * EPT
    pltpu.sync_copy(a_hbm, a_v)
    pltpu.sync_copy(M_hbm, m_v)

    def zden(i, carry):
        den_v[pl.ds(i * 16, 16)] = jnp.zeros((16,), _f32)
        return carry

    lax.fori_loop(0, NPAD // 16, zden, 0)
    for j in range(16):
        for cb in range(D // 16):
            rows_v[j, pl.ds(cb * 16, 16)] = jnp.zeros((16,), _f32)

    def zsh(i, carry):
        pltpu.sync_copy(rows_v, S_sh.at[pl.ds(sid * ROWS_PT + i * 16, 16)])
        return carry

    lax.fori_loop(0, ROWS_PT // 16, zsh, 0)
    plsc.subcore_barrier()

    def chunk_body(k, carry):
        pltpu.sync_copy(src_hbm.at[pl.ds(base + k * ECHUNK, ECHUNK)], s_v)
        pltpu.sync_copy(dst_hbm.at[pl.ds(base + k * ECHUNK, ECHUNK)], d_v)

        def edge_body(i, carry2):
            s = s_v[pl.ds(i * 16, 16)]
            d = d_v[pl.ds(i * 16, 16)]
            g = plsc.load_gather(a_v, [s])
            m = plsc.load_gather(m_v, [d])
            ok = s != d
            ex = jnp.where(ok, jnp.exp(g - m), 0.0)
            plsc.addupdate_scatter(den_v, [d], ex)
            pltpu.async_copy(xs_hbm.at[s], rows_v, sem).wait()
            for j in range(16):
                e = ex[j]
                for cb in range(D // 16):
                    srow_v[j, pl.ds(cb * 16, 16)] = rows_v[j, pl.ds(cb * 16, 16)] * e
            pltpu.sync_copy(srow_v, S_sh.at[d], add=True)
            return carry2

        lax.fori_loop(0, ECHUNK // 16, edge_body, 0)
        return carry

    lax.fori_loop(0, EPT // ECHUNK, chunk_body, 0)
    pltpu.sync_copy(den_v, den_hbm.at[wid])
    plsc.subcore_barrier()
    pltpu.sync_copy(S_sh.at[pl.ds(sid * ROWS_PT, ROWS_PT)],
                    S_hbm.at[cid, pl.ds(sid * ROWS_PT, ROWS_PT)])


def kernel(x, edge_index, W0, b0, lin0, W1, b1, lin1, c0, c1, c2):
    n = x.shape[0]
    src = edge_index[0]
    dst = edge_index[1]
    loops = jnp.arange(n, dtype=src.dtype)
    src_j = jnp.concatenate([src, loops])
    dst_j = jnp.concatenate([dst, loops])
    edge_mask = jnp.concatenate([src != dst, jnp.ones((n,), dtype=jnp.bool_)])

    cin0 = jax.nn.softplus(c0)
    cin1 = jax.nn.softplus(c1)
    xh = jnp.concatenate([jnp.zeros((n, 1), x.dtype), x], axis=1)
    xh = _r_proj(_r_expmap0(_r_proj_tan0(xh), cin0), cin0)

    res0 = _r_prelude(xh, W0, b0, cin0)
    x_tan0 = _r_logmap0(res0, cin0)
    xs0 = x_tan0 @ lin0.T
    a0 = jnp.sum(xs0, axis=-1)

    a0_pad = jnp.zeros((NPAD,), _f32).at[:n].set(a0)
    srcp = jnp.concatenate([src, loops, jnp.zeros((EJP - EJ,), src.dtype)])
    dstp = jnp.concatenate([dst, loops, jnp.zeros((EJP - EJ,), dst.dtype)])

    ae_pad, mp0 = _get_sc_ae()(srcp, dstp, a0_pad)
    amax0 = _tc_comb2(mp0)
    sub0 = _get_sc_edgesub()(dstp, ae_pad, amax0)
    ex0 = jnp.exp(sub0[:EJ])
    denom0 = jax.ops.segment_sum(ex0, dst_j, num_segments=n)
    den_pad = jnp.zeros((NPAD,), _f32).at[:n].set(denom0)
    deng0 = _get_sc_edgegather()(dstp, den_pad)
    alpha0 = ex0 / (deng0[:EJ] + 1e-16)
    alpha0_pad = jnp.concatenate([alpha0, jnp.zeros((EJP - EJ,), _f32)])
    xs0_pad = jnp.zeros((NPAD, D), _f32).at[:n].set(xs0)
    upd0 = _get_sc_upd()(srcp, alpha0_pad, xs0_pad)
    agg0 = jax.ops.segment_sum(upd0[:EJ], dst_j, num_segments=n)

    out0 = _r_proj(_r_expmap(res0, agg0, cin0), cin0)
    out0 = jax.nn.relu(_r_logmap0(out0, cin0))
    out0 = _r_proj_tan0(out0)
    xh = _r_proj(_r_expmap0(out0, cin1), cin1)
    res1 = _r_prelude(xh, W1, b1, cin1)

    cin2 = jax.nn.softplus(c2)
    sq1 = 1.0 / jnp.sqrt(cin1[0])
    sq2 = 1.0 / jnp.sqrt(cin2[0])
    s1 = jnp.reshape(sq1, (1, 1))
    s12 = jnp.stack([sq1, sq2]).reshape(1, 2)

    sqrtK1 = jnp.sqrt(1.0 / cin1)
    normu = jnp.minimum(_r_minkowski_norm(res1), MAX_NORM)
    theta1 = jnp.maximum(normu / sqrtK1, MIN_NORM)[:, 0]

    res1_pad = jnp.zeros((NPAD, D), _f32).at[:n].set(res1)
    th_pad = jnp.ones((NPAD,), _f32).at[:n].set(theta1)
    xs1, a1 = _tc_xs(res1_pad, lin1.T, s1)

    sc_max = _get_sc_max()
    sc_heavy = _get_sc_heavy()
    mp = sc_max(src, dst, a1)
    M1 = _tc_comb(mp, a1)
    S, dp = sc_heavy(src, dst, a1, M1, xs1)

    out = _tc_tail(res1_pad, xs1, a1, M1, S, dp, th_pad, s12)
    return out[:n]
